# baseline (speedup 1.0000x reference)
import functools

import jax
import jax.numpy as jnp
from jax import lax
from jax.experimental import pallas as pl
from jax.experimental.pallas import tpu as pltpu
from jax.experimental.pallas import tpu_sc as plsc

B = 320000
D = 128
C = 128
NUM_CHUNKS = B // C
L = 16


@functools.partial(jax.jit, static_argnums=())
def _score_edges(x_user, x_movie, u_idx, m_idx):
    mesh = plsc.VectorSubcoreMesh(core_axis_name="c", subcore_axis_name="s")

    @functools.partial(
        pl.kernel,
        mesh=mesh,
        out_type=jax.ShapeDtypeStruct((B,), jnp.float32),
        scratch_types=[
            pltpu.VMEM((C,), jnp.int32),
            pltpu.VMEM((C,), jnp.int32),
            pltpu.VMEM((C, D), jnp.float32),
            pltpu.VMEM((C, D), jnp.float32),
            pltpu.VMEM((C,), jnp.float32),
            pltpu.SemaphoreType.DMA,
            pltpu.SemaphoreType.DMA,
        ],
        compiler_params=pltpu.CompilerParams(needs_layout_passes=False),
    )
    def k(u_hbm, m_hbm, uidx_hbm, midx_hbm, out_hbm,
          uidx_v, midx_v, urows_v, mrows_v, out_v, sem_u, sem_m):
        wid = lax.axis_index("c") * 16 + lax.axis_index("s")
        n_workers = 32
        n_chunks_w = jnp.where(wid < NUM_CHUNKS % n_workers,
                               NUM_CHUNKS // n_workers + 1,
                               NUM_CHUNKS // n_workers)

        def chunk_body(i, _):
            base = (wid + i * n_workers) * C
            pltpu.sync_copy(uidx_hbm.at[pl.ds(base, C)], uidx_v)
            pltpu.sync_copy(midx_hbm.at[pl.ds(base, C)], midx_v)
            cp_u = pltpu.async_copy(u_hbm.at[uidx_v], urows_v, sem_u)
            cp_m = pltpu.async_copy(m_hbm.at[midx_v], mrows_v, sem_m)
            cp_u.wait()
            cp_m.wait()

            lane = jax.lax.iota(jnp.int32, L)
            for g in range(C // L):
                rows = lane + g * L

                def col_body(j, acc):
                    cols = jnp.full((L,), j, jnp.int32)
                    u = plsc.load_gather(urows_v, [rows, cols])
                    m = plsc.load_gather(mrows_v, [rows, cols])
                    return acc + u * m

                acc = lax.fori_loop(0, D, col_body,
                                    jnp.zeros((L,), jnp.float32))
                out_v[pl.ds(g * L, L)] = acc
            pltpu.sync_copy(out_v, out_hbm.at[pl.ds(base, C)])
            return 0

        lax.fori_loop(0, n_chunks_w, chunk_body, 0)

    return k(x_user, x_movie, u_idx, m_idx)


def kernel(x_user, x_movie, edge_label_index):
    idx = edge_label_index.astype(jnp.int32)
    return _score_edges(x_user, x_movie, idx[0], idx[1])


# speedup vs baseline: 1.1866x; 1.1866x over previous
import functools

import jax
import jax.numpy as jnp
from jax import lax
from jax.experimental import pallas as pl
from jax.experimental.pallas import tpu as pltpu
from jax.experimental.pallas import tpu_sc as plsc

B = 320000
D = 128
C = 128
L = 16
NW = 32
BW = B // NW
NFULL = BW // C
TAIL = BW - NFULL * C


@jax.jit
def _impl(x_user, x_movie, u_idx, m_idx):
    mesh = plsc.VectorSubcoreMesh(core_axis_name="c", subcore_axis_name="s")

    @functools.partial(
        pl.kernel,
        mesh=mesh,
        out_type=jax.ShapeDtypeStruct((B,), jnp.float32),
        scratch_types=[
            pltpu.VMEM((BW,), jnp.int32),
            pltpu.VMEM((BW,), jnp.int32),
            pltpu.VMEM((C, D), jnp.float32),
            pltpu.VMEM((C, D), jnp.float32),
            pltpu.VMEM((C, D), jnp.float32),
            pltpu.VMEM((C, D), jnp.float32),
            pltpu.VMEM((TAIL, D), jnp.float32),
            pltpu.VMEM((TAIL, D), jnp.float32),
            pltpu.VMEM((BW,), jnp.float32),
            pltpu.SemaphoreType.DMA,
            pltpu.SemaphoreType.DMA,
        ],
        compiler_params=pltpu.CompilerParams(needs_layout_passes=False),
    )
    def k(u_hbm, m_hbm, uidx_hbm, midx_hbm, out_hbm,
          uidx_v, midx_v, u0, m0, u1, m1, ut, mt, out_v, sem_u, sem_m):
        wid = lax.axis_index("c") * 16 + lax.axis_index("s")
        wbase = wid * BW
        pltpu.sync_copy(uidx_hbm.at[pl.ds(wbase, BW)], uidx_v)
        pltpu.sync_copy(midx_hbm.at[pl.ds(wbase, BW)], midx_v)

        lane = lax.iota(jnp.int32, L)

        def fire(i, ub, mb):
            pltpu.async_copy(u_hbm.at[uidx_v.at[pl.ds(i * C, C)]], ub, sem_u)
            pltpu.async_copy(m_hbm.at[midx_v.at[pl.ds(i * C, C)]], mb, sem_m)

        def drain(i, ub, mb):
            pltpu.make_async_copy(
                u_hbm.at[uidx_v.at[pl.ds(i * C, C)]], ub, sem_u).wait()
            pltpu.make_async_copy(
                m_hbm.at[midx_v.at[pl.ds(i * C, C)]], mb, sem_m).wait()

        def compute(i, ub, mb, n_rows):
            for g in range(n_rows // L):
                rows = lane + g * L

                def col_body(jj, acc):
                    for t in range(8):
                        cols = jnp.full((L,), jj * 8 + t, jnp.int32)
                        acc = acc + (plsc.load_gather(ub, [rows, cols]) *
                                     plsc.load_gather(mb, [rows, cols]))
                    return acc

                acc = lax.fori_loop(0, D // 8, col_body,
                                    jnp.zeros((L,), jnp.float32))
                out_v[pl.ds(i * C + g * L, L)] = acc

        fire(0, u0, m0)

        def pair_body(kk, _):
            i0 = 2 * kk
            drain(i0, u0, m0)
            fire(i0 + 1, u1, m1)
            compute(i0, u0, m0, C)
            drain(i0 + 1, u1, m1)
            fire(i0 + 2, u0, m0)
            compute(i0 + 1, u1, m1, C)
            return 0

        lax.fori_loop(0, NFULL // 2 - 1, pair_body, 0)

        drain(NFULL - 2, u0, m0)
        fire(NFULL - 1, u1, m1)
        compute(NFULL - 2, u0, m0, C)
        drain(NFULL - 1, u1, m1)
        pltpu.async_copy(
            u_hbm.at[uidx_v.at[pl.ds(NFULL * C, TAIL)]], ut, sem_u)
        pltpu.async_copy(
            m_hbm.at[midx_v.at[pl.ds(NFULL * C, TAIL)]], mt, sem_m)
        compute(NFULL - 1, u1, m1, C)
        pltpu.make_async_copy(
            u_hbm.at[uidx_v.at[pl.ds(NFULL * C, TAIL)]], ut, sem_u).wait()
        pltpu.make_async_copy(
            m_hbm.at[midx_v.at[pl.ds(NFULL * C, TAIL)]], mt, sem_m).wait()
        compute(NFULL, ut, mt, TAIL)

        pltpu.sync_copy(out_v, out_hbm.at[pl.ds(wbase, BW)])

    return k(x_user, x_movie, u_idx, m_idx)


def kernel(x_user, x_movie, edge_label_index):
    idx = edge_label_index.astype(jnp.int32)
    return _impl(x_user, x_movie, idx[0], idx[1])
